# baseline (speedup 1.0000x reference)
import dataclasses
import functools

import jax
import jax.numpy as jnp
from jax import lax
from jax.experimental import pallas as pl
from jax.experimental.pallas import tpu as pltpu
from jax.experimental.pallas import tpu_sc as plsc

B = 16384
D = 128
T_ROWS = 352
NC = 2
NS = 16
NW = NC * NS
BPW = B // NW
CHUNK = 128
NCHUNK = BPW // CHUNK
HI = lax.Precision.HIGHEST


def _mm3(a, b):
    a_hi = a.astype(jnp.bfloat16)
    a_lo = (a - a_hi.astype(jnp.float32)).astype(jnp.bfloat16)
    b_hi = b.astype(jnp.bfloat16)
    b_lo = (b - b_hi.astype(jnp.float32)).astype(jnp.bfloat16)
    f32 = jnp.float32
    return (jnp.dot(a_hi, b_hi, preferred_element_type=f32)
            + (jnp.dot(a_hi, b_lo, preferred_element_type=f32)
               + jnp.dot(a_lo, b_hi, preferred_element_type=f32)))


def _prep_body(xt_ref, aug_ref, mag_ref, cls_ref, w0_ref, t_ref, idx_ref):
    pa = jnp.dot(aug_ref[...], w0_ref[0:128, :], precision=HI)
    pm = jnp.dot(mag_ref[...], w0_ref[128:256, :], precision=HI)
    pc = jnp.dot(cls_ref[...], w0_ref[256:384, :], precision=HI)
    ra = lax.broadcasted_iota(jnp.int32, (250, 25), 0) // 10
    ca = lax.broadcasted_iota(jnp.int32, (250, 25), 1)
    ea = (ra == ca).astype(jnp.float32)
    rm = lax.broadcasted_iota(jnp.int32, (250, 10), 0) % 10
    cm = lax.broadcasted_iota(jnp.int32, (250, 10), 1)
    em = (rm == cm).astype(jnp.float32)
    pair = 0.5 * (jnp.dot(ea, pa, precision=HI) + jnp.dot(em, pm, precision=HI))
    t_ref[...] = jnp.concatenate(
        [pair, pc, jnp.zeros((2, D), jnp.float32)], axis=0)
    x0 = xt_ref[0:1, :]
    x1 = xt_ref[1:2, :]
    x2 = xt_ref[2:3, :]
    x3 = xt_ref[3:4, :]
    x4 = xt_ref[4:5, :]
    ia = 10 * jnp.clip(x0, 0, 24) + jnp.clip(x1, 0, 9)
    ib = 10 * jnp.clip(x2, 0, 24) + jnp.clip(x3, 0, 9)
    ic = 250 + jnp.clip(x4, 0, 99)
    idx_ref[...] = jnp.concatenate(
        [ia, ib, ic, jnp.zeros((5, B), jnp.int32)], axis=0)


def _sc_gather_sum(t_hbm, idx_hbm, out_hbm,
                   ia_v, ib_v, ic_v,
                   ga0, gb0, gc0, ga1, gb1, gc1,
                   sa0, sb0, sc0, sa1, sb1, sc1, so0, so1):
    cid = lax.axis_index("c")
    sid = lax.axis_index("s")
    wid = cid * NS + sid
    base = wid * BPW
    bufs = ((ga0, gb0, gc0, sa0, sb0, sc0, so0),
            (ga1, gb1, gc1, sa1, sb1, sc1, so1))

    da = pltpu.async_copy(idx_hbm.at[0, pl.ds(base, BPW)], ia_v, sa0)
    db = pltpu.async_copy(idx_hbm.at[1, pl.ds(base, BPW)], ib_v, sb0)
    dc = pltpu.async_copy(idx_hbm.at[2, pl.ds(base, BPW)], ic_v, sc0)
    da.wait()
    db.wait()
    dc.wait()

    def issue_gathers(c):
        ga, gb, gc, sa, sb, sc, _ = bufs[c % 2]
        off = c * CHUNK
        return (pltpu.async_copy(t_hbm.at[ia_v.at[pl.ds(off, CHUNK)]], ga, sa),
                pltpu.async_copy(t_hbm.at[ib_v.at[pl.ds(off, CHUNK)]], gb, sb),
                pltpu.async_copy(t_hbm.at[ic_v.at[pl.ds(off, CHUNK)]], gc, sc))

    pend_g = issue_gathers(0)
    pend_out = (None, None)
    for c in range(NCHUNK):
        ga, gb, gc, _, _, _, so = bufs[c % 2]
        for d in pend_g:
            d.wait()
        if c + 1 < NCHUNK:
            prev_out = pend_out[(c + 1) % 2]
            if prev_out is not None:
                prev_out.wait()
            pend_g = issue_gathers(c + 1)

        @pl.loop(0, CHUNK)
        def _(r):
            for g8 in range(8):
                slc = (r, pl.ds(g8 * 16, 16))
                ga.at[slc][...] = (
                    ga.at[slc][...] + gb.at[slc][...] + gc.at[slc][...])

        out_d = pltpu.async_copy(
            ga, out_hbm.at[pl.ds(base + c * CHUNK, CHUNK)], so)
        pend_out = (out_d, pend_out[1]) if c % 2 == 0 else (pend_out[0], out_d)
    for d in pend_out:
        if d is not None:
            d.wait()


def _mlp_body(pre_ref, b0_ref, w1_ref, b1_ref, w2_ref, b2_ref, wout_ref,
              bout_ref, y_ref):
    h = jnp.maximum(pre_ref[...] + b0_ref[...], 0.0)
    h = jnp.maximum(_mm3(h, w1_ref[...]) + b1_ref[...], 0.0)
    h = jnp.maximum(_mm3(h, w2_ref[...]) + b2_ref[...], 0.0)
    y_ref[...] = _mm3(h, wout_ref[...]) + bout_ref[...]


@jax.jit
def kernel(x, aug_table, mag_table, cls_table, W0, b0, W1, b1, W2, b2, Wout,
           bout):
    xt = jnp.zeros((8, B), jnp.int32).at[0:5, :].set(x.T.astype(jnp.int32))

    t_tab, idx = pl.pallas_call(
        _prep_body,
        grid=(1,),
        in_specs=[
            pl.BlockSpec((8, B), lambda i: (0, 0)),
            pl.BlockSpec((25, D), lambda i: (0, 0)),
            pl.BlockSpec((10, D), lambda i: (0, 0)),
            pl.BlockSpec((100, D), lambda i: (0, 0)),
            pl.BlockSpec((384, D), lambda i: (0, 0)),
        ],
        out_specs=[
            pl.BlockSpec((T_ROWS, D), lambda i: (0, 0)),
            pl.BlockSpec((8, B), lambda i: (0, 0)),
        ],
        out_shape=[
            jax.ShapeDtypeStruct((T_ROWS, D), jnp.float32),
            jax.ShapeDtypeStruct((8, B), jnp.int32),
        ],
    )(xt, aug_table, mag_table, cls_table, W0)

    sc_params = pltpu.CompilerParams()
    if "needs_layout_passes" in pltpu.CompilerParams.__dataclass_fields__:
        sc_params = dataclasses.replace(sc_params, needs_layout_passes=False)
    sc_fn = functools.partial(
        pl.kernel,
        out_type=jax.ShapeDtypeStruct((B, D), jnp.float32),
        compiler_params=sc_params,
        mesh=plsc.VectorSubcoreMesh(core_axis_name="c", subcore_axis_name="s"),
        scratch_types=[
            pltpu.VMEM((BPW,), jnp.int32),
            pltpu.VMEM((BPW,), jnp.int32),
            pltpu.VMEM((BPW,), jnp.int32),
            pltpu.VMEM((CHUNK, D), jnp.float32),
            pltpu.VMEM((CHUNK, D), jnp.float32),
            pltpu.VMEM((CHUNK, D), jnp.float32),
            pltpu.VMEM((CHUNK, D), jnp.float32),
            pltpu.VMEM((CHUNK, D), jnp.float32),
            pltpu.VMEM((CHUNK, D), jnp.float32),
            pltpu.SemaphoreType.DMA,
            pltpu.SemaphoreType.DMA,
            pltpu.SemaphoreType.DMA,
            pltpu.SemaphoreType.DMA,
            pltpu.SemaphoreType.DMA,
            pltpu.SemaphoreType.DMA,
            pltpu.SemaphoreType.DMA,
            pltpu.SemaphoreType.DMA,
        ],
    )(_sc_gather_sum)
    pre = sc_fn(t_tab, idx)

    y = pl.pallas_call(
        _mlp_body,
        grid=(B // 1024,),
        in_specs=[
            pl.BlockSpec((1024, D), lambda i: (i, 0)),
            pl.BlockSpec((1, D), lambda i: (0, 0)),
            pl.BlockSpec((D, D), lambda i: (0, 0)),
            pl.BlockSpec((1, D), lambda i: (0, 0)),
            pl.BlockSpec((D, D), lambda i: (0, 0)),
            pl.BlockSpec((1, D), lambda i: (0, 0)),
            pl.BlockSpec((D, 1), lambda i: (0, 0)),
            pl.BlockSpec((1, 1), lambda i: (0, 0)),
        ],
        out_specs=pl.BlockSpec((1024, 1), lambda i: (i, 0)),
        out_shape=jax.ShapeDtypeStruct((B, 1), jnp.float32),
    )(pre, b0.reshape(1, D), W1, b1.reshape(1, D), W2, b2.reshape(1, D),
      Wout, bout.reshape(1, 1))
    return y
